# baseline (device time: 61492 ns/iter reference)
import jax
import jax.numpy as jnp
from jax import lax
from jax.experimental import pallas as pl
from jax.experimental.pallas import tpu as pltpu

N_DEV = 4
SQ = 1024
SKV = 1024
H_PER = 8
DH = 128
D_MODEL = 1024
D_HID = H_PER * DH
SCALE = 0.08838834764831843
BLK = 64
NBLK = SQ // BLK

GROUP_BLOCKS = [[qb for qb in range(NBLK) if qb % 3 == r] for r in range(3)]
PERM_BLOCKS = GROUP_BLOCKS[0] + GROUP_BLOCKS[1] + GROUP_BLOCKS[2]
NB = [len(b) for b in GROUP_BLOCKS]
SZ = [n * BLK for n in NB]
GS = [0, SZ[0], SZ[0] + SZ[1]]
NGR = 3


def kernel(x, Wq, K_ext, V_ext, Wo):
    x2 = x.reshape(SQ, D_MODEL)
    k2 = K_ext.reshape(SKV, D_HID)
    v2 = V_ext.reshape(SKV, D_HID)

    def body(x_ref, wq_ref, k_ref, v_ref, wo_ref, out_ref,
             ctx_ref, xbuf, kbuf, vbuf, sbuf, rbuf, wq_vmem, wo_vmem,
             copy_sems, send_sems, recv_sems):
        my = lax.axis_index("i")
        a_part = my + 1 - 2 * lax.rem(my, 2)
        b_part = 3 - my

        col0 = pl.multiple_of(my * D_HID, D_HID)
        wq_copy = pltpu.make_async_copy(
            wq_ref.at[:, pl.ds(col0, D_HID)], wq_vmem, copy_sems.at[0])
        wo_copy = pltpu.make_async_copy(
            wo_ref.at[pl.ds(col0, D_HID), :], wo_vmem, copy_sems.at[1])
        wq_copy.start()
        wo_copy.start()

        barrier_sem = pltpu.get_barrier_semaphore()
        for nbr in (a_part, b_part):
            pl.semaphore_signal(
                barrier_sem, inc=1,
                device_id=(nbr,), device_id_type=pl.DeviceIdType.MESH,
            )
        pl.semaphore_wait(barrier_sem, 2)

        for j, qb in enumerate(PERM_BLOCKS):
            xbuf[j * BLK:(j + 1) * BLK] = (
                x_ref[qb * BLK:(qb + 1) * BLK].astype(jnp.bfloat16))
            kbuf[j * BLK:(j + 1) * BLK] = (
                k_ref[qb * BLK:(qb + 1) * BLK].astype(jnp.bfloat16))
            vbuf[j * BLK:(j + 1) * BLK] = (
                v_ref[qb * BLK:(qb + 1) * BLK].astype(jnp.bfloat16))
        wq_copy.wait()
        wq = wq_vmem[...].astype(jnp.bfloat16)
        q = lax.dot_general(
            xbuf[...], wq, (((1,), (0,)), ((), ())),
            preferred_element_type=jnp.float32,
        )
        q = (q * SCALE).astype(jnp.bfloat16)
        wo_copy.wait()
        wo = wo_vmem[...].astype(jnp.bfloat16)

        def compute_group(r):
            s0, n, nb = GS[r], SZ[r], NB[r]
            g = (3 - r) % 3
            gs, gn = GS[g], SZ[g]
            for h in range(H_PER):
                hs, he = h * DH, (h + 1) * DH
                qh = q[s0:s0 + n, hs:he]
                s1 = lax.dot_general(
                    qh, kbuf[gs:gs + gn, hs:he], (((1,), (1,)), ((), ())),
                    preferred_element_type=jnp.float32,
                )
                e1 = jnp.exp(s1)
                rsum = jnp.sum(e1, axis=-1, keepdims=True)
                ctx = lax.dot_general(
                    e1.astype(jnp.bfloat16), vbuf[gs:gs + gn, hs:he],
                    (((1,), (0,)), ((), ())),
                    preferred_element_type=jnp.float32,
                )
                if r != 0:
                    s0p = lax.dot_general(
                        qh, kbuf[0:BLK, hs:he], (((1,), (1,)), ((), ())),
                        preferred_element_type=jnp.float32,
                    )
                    e0 = jnp.exp(s0p)
                    rsum += jnp.sum(e0, axis=-1, keepdims=True)
                    ctx += lax.dot_general(
                        e0.astype(jnp.bfloat16), vbuf[0:BLK, hs:he],
                        (((1,), (0,)), ((), ())),
                        preferred_element_type=jnp.float32,
                    )
                    q3 = qh.reshape(nb, BLK, DH)
                    k3 = kbuf[s0:s0 + n, hs:he].reshape(nb, BLK, DH)
                    sd = lax.dot_general(
                        q3, k3, (((2,), (2,)), ((0,), (0,))),
                        preferred_element_type=jnp.float32,
                    )
                    ed = jnp.exp(sd)
                    rsum += jnp.sum(ed, axis=-1).reshape(n, 1)
                    v3 = vbuf[s0:s0 + n, hs:he].reshape(nb, BLK, DH)
                    cd = lax.dot_general(
                        ed.astype(jnp.bfloat16), v3,
                        (((2,), (1,)), ((0,), (0,))),
                        preferred_element_type=jnp.float32,
                    )
                    ctx += cd.reshape(n, DH)
                ctx_ref[:n, hs:he] = (ctx * (1.0 / rsum)).astype(jnp.bfloat16)
            return lax.dot_general(
                ctx_ref[:n, :], wo, (((1,), (0,)), ((), ())),
                preferred_element_type=jnp.float32,
            )

        def exchange(phase, c, target):
            rows = pl.ds(GS[c], SZ[c])
            rdma = pltpu.make_async_remote_copy(
                src_ref=sbuf.at[phase, rows],
                dst_ref=rbuf.at[phase, rows],
                send_sem=send_sems.at[phase, c],
                recv_sem=recv_sems.at[phase, c],
                device_id=(target,),
                device_id_type=pl.DeviceIdType.MESH,
            )
            rdma.start()
            return rdma

        def partner(phase, c):
            return a_part if (c + phase) % 2 == 0 else b_part

        p1 = [None] * NGR
        p2 = [None] * NGR

        def finish_phase1(c):
            s0, n = GS[c], SZ[c]
            p1[c].wait()
            sbuf[1, s0:s0 + n] = sbuf[0, s0:s0 + n] + rbuf[0, s0:s0 + n]
            p2[c] = exchange(1, c, partner(1, c))

        def finish_phase2(c):
            s0, n = GS[c], SZ[c]
            p2[c].wait()
            final = (sbuf[1, s0:s0 + n].astype(jnp.float32)
                     + rbuf[1, s0:s0 + n].astype(jnp.float32))
            for j, qb in enumerate(GROUP_BLOCKS[c]):
                out_ref[qb * BLK:(qb + 1) * BLK] = (
                    final[j * BLK:(j + 1) * BLK])

        for c in range(NGR):
            s0, n = GS[c], SZ[c]
            sbuf[0, s0:s0 + n] = compute_group(c).astype(jnp.bfloat16)
            p1[c] = exchange(0, c, partner(0, c))
            if c >= 1:
                finish_phase1(c - 1)
            if c >= 2:
                finish_phase2(c - 2)
        finish_phase1(NGR - 1)
        finish_phase2(NGR - 2)
        finish_phase2(NGR - 1)

    out = pl.pallas_call(
        body,
        out_shape=jax.ShapeDtypeStruct((SQ, D_MODEL), jnp.float32),
        in_specs=[
            pl.BlockSpec(memory_space=pltpu.VMEM),
            pl.BlockSpec(memory_space=pl.ANY),
            pl.BlockSpec(memory_space=pltpu.VMEM),
            pl.BlockSpec(memory_space=pltpu.VMEM),
            pl.BlockSpec(memory_space=pl.ANY),
        ],
        out_specs=pl.BlockSpec(memory_space=pltpu.VMEM),
        scratch_shapes=[
            pltpu.VMEM((SZ[0], D_HID), jnp.bfloat16),
            pltpu.VMEM((SQ, D_MODEL), jnp.bfloat16),
            pltpu.VMEM((SKV, D_HID), jnp.bfloat16),
            pltpu.VMEM((SKV, D_HID), jnp.bfloat16),
            pltpu.VMEM((2, SQ, D_MODEL), jnp.bfloat16),
            pltpu.VMEM((2, SQ, D_MODEL), jnp.bfloat16),
            pltpu.VMEM((D_MODEL, D_HID), jnp.float32),
            pltpu.VMEM((D_HID, D_MODEL), jnp.float32),
            pltpu.SemaphoreType.DMA((2,)),
            pltpu.SemaphoreType.DMA((2, NGR)),
            pltpu.SemaphoreType.DMA((2, NGR)),
        ],
        compiler_params=pltpu.CompilerParams(
            collective_id=0, vmem_limit_bytes=100 * 1024 * 1024,
        ),
    )(x2, Wq, k2, v2, Wo)
    return out.reshape(1, SQ, D_MODEL)
